# baseline (speedup 1.0000x reference)
import functools

import jax
import jax.numpy as jnp
from jax import lax
from jax.experimental import pallas as pl
from jax.experimental.pallas import tpu as pltpu
from jax.experimental.pallas import tpu_sc as plsc

DEPTH = 128
D = 768
TOTAL = 32 * 1024
NW = 32
RPW = TOTAL // NW
C = 8
NCHUNK = RPW // C
NSET = 2
L = 16
MAGIC = 8388608.0


def _round_f32(x):
    return (x + MAGIC) - MAGIC


_mesh = plsc.VectorSubcoreMesh(
    core_axis_name="c", subcore_axis_name="s", num_cores=2, num_subcores=16
)


@functools.partial(
    pl.kernel,
    out_type=jax.ShapeDtypeStruct((TOTAL * D,), jnp.float32),
    mesh=_mesh,
    compiler_params=pltpu.CompilerParams(needs_layout_passes=False),
    scratch_types=[
        pltpu.VMEM((RPW,), jnp.float32),
        pltpu.VMEM((RPW,), jnp.float32),
        pltpu.VMEM((RPW,), jnp.int32),
        pltpu.VMEM((DEPTH * D // 2,), jnp.int32),
        pltpu.VMEM((DEPTH * D // 2,), jnp.int32),
        [pltpu.VMEM((C * D,), jnp.float32)] * NSET,
        [pltpu.VMEM((C * D,), jnp.float32)] * NSET,
        [pltpu.SemaphoreType.DMA] * NSET,
        [pltpu.SemaphoreType.DMA] * NSET,
        pltpu.SemaphoreType.DMA,
    ],
)
def _sc_kernel(in_hbm, rf, rt, cf, ct, row_tab, col_tab, out_hbm,
               pa, pb, pidx_v, row_v, col_v,
               in_buf, out_buf, isem, osem, tsem):
    cid = lax.axis_index("c")
    sid = lax.axis_index("s")
    wid = sid * 2 + cid
    base = wid * RPW

    t0 = pltpu.async_copy(row_tab, row_v, tsem)
    t1 = pltpu.async_copy(col_tab, col_v, tsem)

    def discretized(pfrom_chunk, pto_chunk):
        a = _round_f32(pfrom_chunk * DEPTH)
        b = _round_f32(pto_chunk * DEPTH)
        s = (a + b).astype(jnp.int32)
        m = s >> 1
        return jnp.minimum(m + ((s & 1) & (m & 1)), DEPTH - 1)

    pltpu.sync_copy(rf.at[pl.ds(base, RPW)], pa)
    pltpu.sync_copy(rt.at[pl.ds(base, RPW)], pb)

    def rbody(i, carry):
        sl = pl.ds(i * L, L)
        pidx_v[sl] = discretized(pa[sl], pb[sl])
        return carry

    lax.fori_loop(0, RPW // L, rbody, 0)

    pltpu.sync_copy(cf.at[pl.ds(base, RPW)], pa)
    pltpu.sync_copy(ct.at[pl.ds(base, RPW)], pb)

    def cbody(i, carry):
        sl = pl.ds(i * L, L)
        pidx_v[sl] = pidx_v[sl] | (discretized(pa[sl], pb[sl]) << 16)
        return carry

    lax.fori_loop(0, RPW // L, cbody, 0)

    t0.wait()
    t1.wait()

    def start_in(j, b):
        pltpu.async_copy(in_hbm.at[pl.ds((base + j * C) * D, C * D)],
                         in_buf[b], isem[b])

    def wait_in(b):
        pltpu.make_async_copy(in_hbm.at[pl.ds(base * D, C * D)], in_buf[b],
                              isem[b]).wait()

    start_in(0, 0)
    start_in(1, 1)

    _IOTA = lax.iota(jnp.int32, L)

    def expand(vi):
        lo = lax.bitcast_convert_type(vi << 16, jnp.float32)
        hi = lax.bitcast_convert_type(vi & (-65536), jnp.float32)
        return lo, hi

    def chunk_pair(g, carry):
        for b in range(NSET):
            j = g * NSET + b
            wait_in(b)

            @pl.when(g > 0)
            def _():
                pltpu.make_async_copy(out_buf[b],
                                      out_hbm.at[pl.ds(base * D, C * D)],
                                      osem[b]).wait()

            def row_body(i, c2):
                row = j * C + i
                pvec = pidx_v[pl.ds(row & ~15, L)]
                p = jnp.max(jnp.where(_IOTA == (row & 15), pvec, 0))
                r = (p & 0xFFFF) * (D // 2)
                c = (p >> 16) * (D // 2)
                o = i * D
                for k in range(D // 32):
                    ra, rb2 = expand(row_v[pl.ds(r + k * L, L)])
                    ca, cb2 = expand(col_v[pl.ds(c + k * L, L)])
                    sl0 = pl.ds(o + k * 32, L)
                    sl1 = pl.ds(o + k * 32 + L, L)
                    out_buf[b][sl0] = in_buf[b][sl0] + ra + ca
                    out_buf[b][sl1] = in_buf[b][sl1] + rb2 + cb2
                return c2

            lax.fori_loop(0, C, row_body, 0)

            pltpu.async_copy(out_buf[b],
                             out_hbm.at[pl.ds((base + j * C) * D, C * D)],
                             osem[b])
            jn = jnp.minimum(j + NSET, NCHUNK - 1)
            start_in(jn, b)
        return carry

    lax.fori_loop(0, NCHUNK // NSET, chunk_pair, 0)

    for b in range(NSET):
        wait_in(b)
        pltpu.make_async_copy(out_buf[b], out_hbm.at[pl.ds(base * D, C * D)],
                              osem[b]).wait()


def _permute_table(tab):
    d = tab.shape[-1]
    t = tab.astype(jnp.bfloat16).reshape(DEPTH, d // 32, 2, 16)
    t = t.transpose(0, 1, 3, 2).reshape(DEPTH * d // 2, 2)
    return jax.lax.bitcast_convert_type(t, jnp.int32)


def kernel(input_ids, row_pos_from, row_pos_to, col_pos_from, col_pos_to,
           row_table, col_table):
    b, p, d = input_ids.shape
    out = _sc_kernel(
        input_ids.reshape(-1),
        row_pos_from.reshape(-1),
        row_pos_to.reshape(-1),
        col_pos_from.reshape(-1),
        col_pos_to.reshape(-1),
        _permute_table(row_table),
        _permute_table(col_table),
    )
    return out.reshape(b, p, d)


# speedup vs baseline: 1.5749x; 1.5749x over previous
import functools

import jax
import jax.numpy as jnp
from jax import lax
from jax.experimental import pallas as pl
from jax.experimental.pallas import tpu as pltpu
from jax.experimental.pallas import tpu_sc as plsc

DEPTH = 128
D = 768
TOTAL = 32 * 1024
NW = 32
RPW = TOTAL // NW
C = 8
NCHUNK = RPW // C
NSET = 2
L = 16
MAGIC = 8388608.0


def _round_f32(x):
    return (x + MAGIC) - MAGIC


_mesh = plsc.VectorSubcoreMesh(
    core_axis_name="c", subcore_axis_name="s", num_cores=2, num_subcores=16
)


@functools.partial(
    pl.kernel,
    out_type=jax.ShapeDtypeStruct((TOTAL * D,), jnp.float32),
    mesh=_mesh,
    compiler_params=pltpu.CompilerParams(needs_layout_passes=False),
    scratch_types=[
        pltpu.VMEM((RPW,), jnp.float32),
        pltpu.VMEM((RPW,), jnp.float32),
        pltpu.VMEM((RPW,), jnp.int32),
        pltpu.VMEM((DEPTH * D // 2,), jnp.int32),
        pltpu.VMEM((DEPTH * D // 2,), jnp.int32),
        [pltpu.VMEM((C * D,), jnp.float32)] * NSET,
        [pltpu.VMEM((C * D,), jnp.float32)] * NSET,
        [pltpu.SemaphoreType.DMA] * NSET,
        [pltpu.SemaphoreType.DMA] * NSET,
        pltpu.SemaphoreType.DMA,
    ],
)
def _sc_kernel(in_hbm, rf, rt, cf, ct, row_tab, col_tab, out_hbm,
               pa, pb, pidx_v, row_v, col_v,
               in_buf, out_buf, isem, osem, tsem):
    cid = lax.axis_index("c")
    sid = lax.axis_index("s")
    wid = sid * 2 + cid
    base = wid * RPW

    t0 = pltpu.async_copy(row_tab, row_v, tsem)
    t1 = pltpu.async_copy(col_tab, col_v, tsem)

    def discretized(pfrom_chunk, pto_chunk):
        a = _round_f32(pfrom_chunk * DEPTH)
        b = _round_f32(pto_chunk * DEPTH)
        s = (a + b).astype(jnp.int32)
        m = s >> 1
        return jnp.minimum(m + ((s & 1) & (m & 1)), DEPTH - 1)

    pltpu.sync_copy(rf.at[pl.ds(base, RPW)], pa)
    pltpu.sync_copy(rt.at[pl.ds(base, RPW)], pb)

    @plsc.parallel_loop(0, RPW // L, unroll=4)
    def rbody(i):
        sl = pl.ds(i * L, L)
        pidx_v[sl] = discretized(pa[sl], pb[sl])

    pltpu.sync_copy(cf.at[pl.ds(base, RPW)], pa)
    pltpu.sync_copy(ct.at[pl.ds(base, RPW)], pb)

    @plsc.parallel_loop(0, RPW // L, unroll=4)
    def cbody(i):
        sl = pl.ds(i * L, L)
        pidx_v[sl] = pidx_v[sl] | (discretized(pa[sl], pb[sl]) << 16)

    t0.wait()
    t1.wait()

    def start_in(j, b):
        pltpu.async_copy(in_hbm.at[pl.ds((base + j * C) * D, C * D)],
                         in_buf[b], isem[b])

    def wait_in(b):
        pltpu.make_async_copy(in_hbm.at[pl.ds(base * D, C * D)], in_buf[b],
                              isem[b]).wait()

    start_in(0, 0)
    start_in(1, 1)

    _IOTA = lax.iota(jnp.int32, L)

    def expand(vi):
        lo = lax.bitcast_convert_type(vi << 16, jnp.float32)
        hi = lax.bitcast_convert_type(vi & (-65536), jnp.float32)
        return lo, hi

    def chunk_pair(g, carry):
        for b in range(NSET):
            j = g * NSET + b
            wait_in(b)

            @pl.when(g > 0)
            def _():
                pltpu.make_async_copy(out_buf[b],
                                      out_hbm.at[pl.ds(base * D, C * D)],
                                      osem[b]).wait()

            @plsc.parallel_loop(0, C, unroll=4)
            def row_body(i):
                row = j * C + i
                pvec = pidx_v[pl.ds(row & ~15, L)]
                p = jnp.max(jnp.where(_IOTA == (row & 15), pvec, 0))
                r = (p & 0xFFFF) * (D // 2)
                c = (p >> 16) * (D // 2)
                o = i * D
                for k in range(D // 32):
                    ra, rb2 = expand(row_v[pl.ds(r + k * L, L)])
                    ca, cb2 = expand(col_v[pl.ds(c + k * L, L)])
                    sl0 = pl.ds(o + k * 32, L)
                    sl1 = pl.ds(o + k * 32 + L, L)
                    out_buf[b][sl0] = in_buf[b][sl0] + ra + ca
                    out_buf[b][sl1] = in_buf[b][sl1] + rb2 + cb2

            pltpu.async_copy(out_buf[b],
                             out_hbm.at[pl.ds((base + j * C) * D, C * D)],
                             osem[b])
            jn = jnp.minimum(j + NSET, NCHUNK - 1)
            start_in(jn, b)
        return carry

    lax.fori_loop(0, NCHUNK // NSET, chunk_pair, 0)

    for b in range(NSET):
        wait_in(b)
        pltpu.make_async_copy(out_buf[b], out_hbm.at[pl.ds(base * D, C * D)],
                              osem[b]).wait()


def _permute_table(tab):
    d = tab.shape[-1]
    t = tab.astype(jnp.bfloat16).reshape(DEPTH, d // 32, 2, 16)
    t = t.transpose(0, 1, 3, 2).reshape(DEPTH * d // 2, 2)
    return jax.lax.bitcast_convert_type(t, jnp.int32)


def kernel(input_ids, row_pos_from, row_pos_to, col_pos_from, col_pos_to,
           row_table, col_table):
    b, p, d = input_ids.shape
    out = _sc_kernel(
        input_ids.reshape(-1),
        row_pos_from.reshape(-1),
        row_pos_to.reshape(-1),
        col_pos_from.reshape(-1),
        col_pos_to.reshape(-1),
        _permute_table(row_table),
        _permute_table(col_table),
    )
    return out.reshape(b, p, d)
